# Initial kernel scaffold; baseline (speedup 1.0000x reference)
import jax
import jax.numpy as jnp
from jax.experimental import pallas as pl


def kernel(x, edge_index, edge_attr, Wt0, Wt1, Wt2, Wt3, bt, W1, b1, W2, b2):
    raise NotImplementedError("write your pallas kernel here")


# speedup vs baseline: 196.5594x; 196.5594x over previous
import functools

import jax
import jax.numpy as jnp
from jax import lax
from jax.experimental import pallas as pl
from jax.experimental.pallas import tpu as pltpu
from jax.experimental.pallas import tpu_sc as plsc

N = 100000
E = 6400000
DIM = 128
NPAD = 102400
NSUB = 16
NCORE = 2
NW = NSUB * NCORE
EPW = E // NW
CHUNK = 4000
SLC = NPAD // NSUB
FBUF = max(CHUNK, SLC)

_sc_mesh = None


def _mesh():
    global _sc_mesh
    if _sc_mesh is None:
        _sc_mesh = plsc.VectorSubcoreMesh(core_axis_name="c", subcore_axis_name="s")
    return _sc_mesh


_SC_PARAMS = pltpu.CompilerParams(needs_layout_passes=False)


def _zero_vmem(buf, nwords):
    z = jnp.zeros((16,), jnp.float32)

    def zz(i, _):
        buf[pl.ds(i * 16, 16)] = z
        return 0

    lax.fori_loop(0, nwords // 16, zz, 0)


def _scatter_chunks(acc, ei, src_hbm, idxbuf, valbuf, wid):
    def chunk(c, _):
        base = pl.multiple_of(wid * EPW + c * CHUNK, 8)
        pltpu.sync_copy(ei.at[pl.ds(E + base, CHUNK)], idxbuf)
        pltpu.sync_copy(src_hbm.at[pl.ds(base, CHUNK)], valbuf.at[pl.ds(0, CHUNK)])

        def inner(i, _):
            o = i * 16
            plsc.addupdate_scatter(acc, [idxbuf[pl.ds(o, 16)]], valbuf[pl.ds(o, 16)])
            return 0

        lax.fori_loop(0, CHUNK // 16, inner, 0)
        return 0

    lax.fori_loop(0, EPW // CHUNK, chunk, 0)


def _reduce_to_partial(acc, accs_hbm, out_hbm, cid, sid):
    wid = cid * NSUB + sid
    pltpu.sync_copy(acc, accs_hbm.at[pl.ds(wid * NPAD, NPAD)])
    plsc.subcore_barrier()

    def rd(t, _):
        src = pl.multiple_of((cid * NSUB + t) * NPAD + sid * SLC, 8)
        pltpu.sync_copy(accs_hbm.at[pl.ds(src, SLC)], acc.at[pl.ds(t * SLC, SLC)])
        return 0

    lax.fori_loop(0, NSUB, rd, 0)

    def red(j, _):
        o = j * 16
        v = acc[pl.ds(o, 16)]
        for t in range(1, NSUB):
            v = v + acc[pl.ds(t * SLC + o, 16)]
        acc[pl.ds(o, 16)] = v
        return 0

    lax.fori_loop(0, SLC // 16, red, 0)
    pltpu.sync_copy(acc.at[pl.ds(0, SLC)],
                    out_hbm.at[pl.ds(cid * NPAD + sid * SLC, SLC)])


@functools.partial(
    pl.kernel,
    out_type=(
        jax.ShapeDtypeStruct((NCORE * NPAD,), jnp.float32),
        jax.ShapeDtypeStruct((NW * NPAD,), jnp.float32),
    ),
    mesh=_mesh(),
    compiler_params=_SC_PARAMS,
    scratch_types=[
        pltpu.VMEM((NPAD,), jnp.float32),
        pltpu.VMEM((CHUNK,), jnp.int32),
        pltpu.VMEM((FBUF,), jnp.float32),
    ],
)
def _deg_kernel(ei, w, degp, accs, acc, idxbuf, valbuf):
    cid = lax.axis_index("c")
    sid = lax.axis_index("s")
    wid = cid * NSUB + sid
    _zero_vmem(acc, NPAD)
    _scatter_chunks(acc, ei, w, idxbuf, valbuf, wid)
    _reduce_to_partial(acc, accs, degp, cid, sid)


def _make_hop_kernel(first):
    @functools.partial(
        pl.kernel,
        out_type=(
            jax.ShapeDtypeStruct((NCORE * NPAD,), jnp.float32),
            jax.ShapeDtypeStruct((NPAD,), jnp.float32),
            jax.ShapeDtypeStruct((E,), jnp.float32),
            jax.ShapeDtypeStruct((NW * NPAD,), jnp.float32),
        ),
        mesh=_mesh(),
        compiler_params=_SC_PARAMS,
        scratch_types=[
            pltpu.VMEM((NPAD,), jnp.float32),
            pltpu.VMEM((CHUNK,), jnp.int32),
            pltpu.VMEM((FBUF,), jnp.float32),
            pltpu.VMEM((FBUF,), jnp.float32),
            pltpu.VMEM_SHARED((NPAD,), jnp.float32),
        ],
    )
    def _hop(pa, nodevec, ei, w, partials, node_out, msg, accs, ubuf, idxbuf,
             valbuf, msgbuf, ubc):
        cid = lax.axis_index("c")
        sid = lax.axis_index("s")
        wid = cid * NSUB + sid
        slc = pl.ds(sid * SLC, SLC)

        pltpu.sync_copy(pa.at[pl.ds(sid * SLC, SLC)], valbuf.at[pl.ds(0, SLC)])
        pltpu.sync_copy(pa.at[pl.ds(NPAD + sid * SLC, SLC)], msgbuf.at[pl.ds(0, SLC)])
        pltpu.sync_copy(nodevec.at[slc], ubuf.at[pl.ds(0, SLC)])

        def pro(j, _):
            o = j * 16
            s = valbuf[pl.ds(o, 16)] + msgbuf[pl.ds(o, 16)]
            v = ubuf[pl.ds(o, 16)]
            if first:
                bits = plsc.bitcast(s, jnp.int32)
                y = plsc.bitcast(jnp.int32(0x5F3759DF) - (bits >> 1), jnp.float32)
                for _ in range(3):
                    y = y * (1.5 - 0.5 * s * y * y)
                dis = jnp.where(s > 0.0, y, 0.0)
                valbuf[pl.ds(o, 16)] = dis
                msgbuf[pl.ds(o, 16)] = dis * v
            else:
                h = v * s
                valbuf[pl.ds(o, 16)] = h
                msgbuf[pl.ds(o, 16)] = v * h
            return 0

        lax.fori_loop(0, SLC // 16, pro, 0)

        @pl.when(cid == 0)
        def _():
            pltpu.sync_copy(valbuf.at[pl.ds(0, SLC)], node_out.at[slc])

        pltpu.sync_copy(msgbuf.at[pl.ds(0, SLC)], ubc.at[slc])
        plsc.subcore_barrier()
        pltpu.sync_copy(ubc, ubuf)
        plsc.subcore_barrier()

        def gchunk(c, _):
            base = pl.multiple_of(wid * EPW + c * CHUNK, 8)
            pltpu.sync_copy(ei.at[pl.ds(base, CHUNK)], idxbuf)
            pltpu.sync_copy(w.at[pl.ds(base, CHUNK)], valbuf.at[pl.ds(0, CHUNK)])

            def inner(i, _):
                o = i * 16
                g = plsc.load_gather(ubuf, [idxbuf[pl.ds(o, 16)]])
                msgbuf[pl.ds(o, 16)] = g * valbuf[pl.ds(o, 16)]
                return 0

            lax.fori_loop(0, CHUNK // 16, inner, 0)
            pltpu.sync_copy(msgbuf.at[pl.ds(0, CHUNK)], msg.at[pl.ds(base, CHUNK)])
            return 0

        lax.fori_loop(0, EPW // CHUNK, gchunk, 0)

        _zero_vmem(ubuf, NPAD)
        _scatter_chunks(ubuf, ei, msg, idxbuf, valbuf, wid)
        _reduce_to_partial(ubuf, accs, partials, cid, sid)

    return _hop


_hop_first = _make_hop_kernel(True)
_hop_next = _make_hop_kernel(False)

BLKR = 2048


def _head_body(v3, dis, s30, s31, wr013, wt3, bt, w1, b1, w2, b2, out):
    h3 = dis[...] * (s30[...] + s31[...])
    z = lax.dot_general(v3[...], wr013[...], (((1,), (0,)), ((), ())),
                        preferred_element_type=jnp.float32)
    z = z + h3 * wt3[...]
    z = jnp.maximum(z + bt[...], 0.0)
    z = lax.dot_general(z, w1[...], (((1,), (1,)), ((), ())),
                        preferred_element_type=jnp.float32)
    z = jnp.maximum(z + b1[...], 0.0)
    y = jnp.sum(z * w2[...], axis=1, keepdims=True) + b2[...]
    out[...] = jnp.maximum(y, 0.0)


def _head(v3, dis, s30, s31, wr013, wt3, bt, w1, b1, w2, b2):
    full = lambda r, c: pl.BlockSpec((r, c), lambda i: (0, 0))
    blk = lambda c: pl.BlockSpec((BLKR, c), lambda i: (i, 0))
    return pl.pallas_call(
        _head_body,
        grid=(NPAD // BLKR,),
        in_specs=[blk(3), blk(1), blk(1), blk(1), full(3, DIM), full(1, DIM),
                  full(1, DIM), full(DIM, DIM), full(1, DIM), full(1, DIM),
                  full(1, 1)],
        out_specs=blk(1),
        out_shape=jax.ShapeDtypeStruct((NPAD, 1), jnp.float32),
    )(v3, dis, s30, s31, wr013, wt3, bt, w1, b1, w2, b2)


def kernel(x, edge_index, edge_attr, Wt0, Wt1, Wt2, Wt3, bt, W1, b1, W2, b2):
    if x.ndim == 1:
        x = x[:, None]
    xp = jnp.pad(x[:, 0].astype(jnp.float32), (0, NPAD - N))
    ei = edge_index.reshape(-1)
    degp, _ = _deg_kernel(ei, edge_attr)
    s1p, dis, _, _ = _hop_first(degp, xp, ei, edge_attr)
    s2p, h1, _, _ = _hop_next(s1p, dis, ei, edge_attr)
    s3p, h2, _, _ = _hop_next(s2p, dis, ei, edge_attr)
    v3 = jnp.stack([xp, h1, h2], axis=1)
    wr013 = jnp.concatenate([Wt0, Wt1, Wt2], axis=1).T
    out = _head(v3, dis[:, None], s3p[:NPAD, None], s3p[NPAD:, None], wr013,
                Wt3.T, bt[None, :], W1, b1[None, :], W2, b2[None, :])
    return out[:N]


# speedup vs baseline: 258.6535x; 1.3159x over previous
import functools

import jax
import jax.numpy as jnp
from jax import lax
from jax.experimental import pallas as pl
from jax.experimental.pallas import tpu as pltpu
from jax.experimental.pallas import tpu_sc as plsc

N = 100000
E = 6400000
DIM = 128
NPAD = 102400
NSUB = 16
NCORE = 2
NW = NSUB * NCORE
EPW = E // NW
CHUNK = 2000
NCH = EPW // CHUNK
SLC = NPAD // NSUB

_sc_mesh = None


def _mesh():
    global _sc_mesh
    if _sc_mesh is None:
        _sc_mesh = plsc.VectorSubcoreMesh(core_axis_name="c", subcore_axis_name="s")
    return _sc_mesh


_SC_PARAMS = pltpu.CompilerParams(needs_layout_passes=False)


def _zero_vmem(buf, nwords):
    z = jnp.zeros((16,), jnp.float32)

    def zz(i, _):
        buf[pl.ds(i * 16, 16)] = z
        return 0

    lax.fori_loop(0, nwords // 16, zz, 0)


def _in_copies(ei, ei_off, src_hbm, idxbuf, valbuf, sem, wid, c, b):
    base = pl.multiple_of(wid * EPW + c * CHUNK, 8)
    sl = pl.ds(b * CHUNK, CHUNK)
    return (
        pltpu.make_async_copy(ei.at[pl.ds(ei_off + base, CHUNK)], idxbuf.at[sl],
                              sem.at[b]),
        pltpu.make_async_copy(src_hbm.at[pl.ds(base, CHUNK)], valbuf.at[sl],
                              sem.at[b]),
    )


def _pipe_scatter(acc, ei, src_hbm, idxbuf, valbuf, sem, wid):
    for d in _in_copies(ei, E, src_hbm, idxbuf, valbuf, sem, wid, 0, 0):
        d.start()

    def body(c, _):
        b = lax.rem(c, 2)

        @pl.when(c + 1 < NCH)
        def _():
            for d in _in_copies(ei, E, src_hbm, idxbuf, valbuf, sem, wid,
                                c + 1, 1 - b):
                d.start()

        for d in _in_copies(ei, E, src_hbm, idxbuf, valbuf, sem, wid, c, b):
            d.wait()

        def inner(i, _):
            o = b * CHUNK + i * 16
            plsc.addupdate_scatter(acc, [idxbuf[pl.ds(o, 16)]], valbuf[pl.ds(o, 16)])
            return 0

        lax.fori_loop(0, CHUNK // 16, inner, 0)
        return 0

    lax.fori_loop(0, NCH, body, 0)


def _pipe_gather(ubuf, ei, w, msg, idxbuf, valbuf, msgbuf, semi, semo, wid):
    def out_copy(c, b):
        base = pl.multiple_of(wid * EPW + c * CHUNK, 8)
        return pltpu.make_async_copy(msgbuf.at[pl.ds(b * CHUNK, CHUNK)],
                                     msg.at[pl.ds(base, CHUNK)], semo.at[b])

    for d in _in_copies(ei, 0, w, idxbuf, valbuf, semi, wid, 0, 0):
        d.start()

    def body(c, _):
        b = lax.rem(c, 2)

        @pl.when(c + 1 < NCH)
        def _():
            for d in _in_copies(ei, 0, w, idxbuf, valbuf, semi, wid, c + 1, 1 - b):
                d.start()

        for d in _in_copies(ei, 0, w, idxbuf, valbuf, semi, wid, c, b):
            d.wait()

        @pl.when(c >= 2)
        def _():
            out_copy(c - 2, b).wait()

        def inner(i, _):
            o = b * CHUNK + i * 16
            g = plsc.load_gather(ubuf, [idxbuf[pl.ds(o, 16)]])
            msgbuf[pl.ds(o, 16)] = g * valbuf[pl.ds(o, 16)]
            return 0

        lax.fori_loop(0, CHUNK // 16, inner, 0)
        out_copy(c, b).start()
        return 0

    lax.fori_loop(0, NCH, body, 0)
    out_copy(NCH - 2, 0).wait()
    out_copy(NCH - 1, 1).wait()


def _reduce_to_partial(acc, accs_hbm, out_hbm, cid, sid, semr):
    wid = cid * NSUB + sid
    pltpu.sync_copy(acc, accs_hbm.at[pl.ds(wid * NPAD, NPAD)])
    plsc.subcore_barrier()

    def rd_copy(t):
        src = pl.multiple_of((cid * NSUB + t) * NPAD + sid * SLC, 8)
        return pltpu.make_async_copy(accs_hbm.at[pl.ds(src, SLC)],
                                     acc.at[pl.ds(t * SLC, SLC)], semr)

    def fire(t, _):
        rd_copy(t).start()
        return 0

    def drain(t, _):
        rd_copy(t).wait()
        return 0

    lax.fori_loop(0, NSUB, fire, 0)
    lax.fori_loop(0, NSUB, drain, 0)

    def red(j, _):
        o = j * 16
        v = [acc[pl.ds(t * SLC + o, 16)] for t in range(NSUB)]
        while len(v) > 1:
            v = [v[i] + v[i + 1] for i in range(0, len(v), 2)]
        acc[pl.ds(o, 16)] = v[0]
        return 0

    lax.fori_loop(0, SLC // 16, red, 0)
    pltpu.sync_copy(acc.at[pl.ds(0, SLC)],
                    out_hbm.at[pl.ds(cid * NPAD + sid * SLC, SLC)])


@functools.partial(
    pl.kernel,
    out_type=(
        jax.ShapeDtypeStruct((NCORE * NPAD,), jnp.float32),
        jax.ShapeDtypeStruct((NW * NPAD,), jnp.float32),
    ),
    mesh=_mesh(),
    compiler_params=_SC_PARAMS,
    scratch_types=[
        pltpu.VMEM((NPAD,), jnp.float32),
        pltpu.VMEM((2 * CHUNK,), jnp.int32),
        pltpu.VMEM((2 * CHUNK,), jnp.float32),
        pltpu.SemaphoreType.DMA((2,)),
        pltpu.SemaphoreType.DMA,
    ],
)
def _deg_kernel(ei, w, degp, accs, acc, idxbuf, valbuf, semi, semr):
    cid = lax.axis_index("c")
    sid = lax.axis_index("s")
    wid = cid * NSUB + sid
    _zero_vmem(acc, NPAD)
    _pipe_scatter(acc, ei, w, idxbuf, valbuf, semi, wid)
    _reduce_to_partial(acc, accs, degp, cid, sid, semr)


def _make_hop_kernel(first):
    @functools.partial(
        pl.kernel,
        out_type=(
            jax.ShapeDtypeStruct((NCORE * NPAD,), jnp.float32),
            jax.ShapeDtypeStruct((NPAD,), jnp.float32),
            jax.ShapeDtypeStruct((E,), jnp.float32),
            jax.ShapeDtypeStruct((NW * NPAD,), jnp.float32),
        ),
        mesh=_mesh(),
        compiler_params=_SC_PARAMS,
        scratch_types=[
            pltpu.VMEM((NPAD,), jnp.float32),
            pltpu.VMEM((2 * CHUNK,), jnp.int32),
            pltpu.VMEM((2 * CHUNK,), jnp.float32),
            pltpu.VMEM((2 * CHUNK,), jnp.float32),
            pltpu.VMEM_SHARED((NPAD,), jnp.float32),
            pltpu.SemaphoreType.DMA((2,)),
            pltpu.SemaphoreType.DMA((2,)),
            pltpu.SemaphoreType.DMA,
        ],
    )
    def _hop(pa, nodevec, ei, w, partials, node_out, msg, accs, ubuf, idxbuf,
             valbuf, msgbuf, ubc, semi, semo, semr):
        cid = lax.axis_index("c")
        sid = lax.axis_index("s")
        wid = cid * NSUB + sid
        slc = pl.ds(sid * SLC, SLC)

        pltpu.sync_copy(pa.at[pl.ds(sid * SLC, SLC)], ubuf.at[pl.ds(0, SLC)])
        pltpu.sync_copy(pa.at[pl.ds(NPAD + sid * SLC, SLC)],
                        ubuf.at[pl.ds(SLC, SLC)])
        pltpu.sync_copy(nodevec.at[slc], ubuf.at[pl.ds(2 * SLC, SLC)])

        def pro(j, _):
            o = j * 16
            v = ubuf[pl.ds(2 * SLC + o, 16)]
            if first:
                h = ubuf[pl.ds(o, 16)]
            else:
                s = ubuf[pl.ds(o, 16)] + ubuf[pl.ds(SLC + o, 16)]
                h = v * s
            ubuf[pl.ds(o, 16)] = h
            ubuf[pl.ds(SLC + o, 16)] = v * h
            return 0

        lax.fori_loop(0, SLC // 16, pro, 0)

        @pl.when(cid == 0)
        def _():
            pltpu.sync_copy(ubuf.at[pl.ds(0, SLC)], node_out.at[slc])

        pltpu.sync_copy(ubuf.at[pl.ds(SLC, SLC)], ubc.at[slc])
        plsc.subcore_barrier()
        pltpu.sync_copy(ubc, ubuf)
        plsc.subcore_barrier()

        _pipe_gather(ubuf, ei, w, msg, idxbuf, valbuf, msgbuf, semi, semo, wid)

        _zero_vmem(ubuf, NPAD)
        _pipe_scatter(ubuf, ei, msg, idxbuf, valbuf, semi, wid)
        _reduce_to_partial(ubuf, accs, partials, cid, sid, semr)

    return _hop


_hop_first = _make_hop_kernel(True)
_hop_next = _make_hop_kernel(False)

BLKR = 2048


def _head_body(x0, h1, h2, dis, s30, s31, wt0, wt1, wt2, wt3, bt, w1, b1,
               w2, b2, out):
    h3 = dis[...] * (s30[...] + s31[...])
    z = x0[...] * wt0[...]
    z = z + h1[...] * wt1[...]
    z = z + h2[...] * wt2[...]
    z = z + h3 * wt3[...]
    z = jnp.maximum(z + bt[...], 0.0)
    z = lax.dot_general(z, w1[...], (((1,), (1,)), ((), ())),
                        precision=lax.Precision.DEFAULT,
                        preferred_element_type=jnp.float32)
    z = jnp.maximum(z + b1[...], 0.0)
    y = lax.dot_general(z, w2[...], (((1,), (0,)), ((), ())),
                        precision=lax.Precision.DEFAULT,
                        preferred_element_type=jnp.float32) + b2[...]
    out[...] = jnp.maximum(y, 0.0)


def _head(x0, h1, h2, dis, s30, s31, wt0, wt1, wt2, wt3, bt, w1, b1, w2, b2):
    full = lambda r, c: pl.BlockSpec((r, c), lambda i: (0, 0))
    blk = lambda c: pl.BlockSpec((BLKR, c), lambda i: (i, 0))
    return pl.pallas_call(
        _head_body,
        grid=(NPAD // BLKR,),
        in_specs=[blk(1), blk(1), blk(1), blk(1), blk(1), blk(1),
                  full(1, DIM), full(1, DIM), full(1, DIM), full(1, DIM),
                  full(1, DIM), full(DIM, DIM), full(1, DIM), full(DIM, 1),
                  full(1, 1)],
        out_specs=blk(1),
        out_shape=jax.ShapeDtypeStruct((NPAD, 1), jnp.float32),
    )(x0, h1, h2, dis, s30, s31, wt0, wt1, wt2, wt3, bt, w1, b1, w2, b2)


def kernel(x, edge_index, edge_attr, Wt0, Wt1, Wt2, Wt3, bt, W1, b1, W2, b2):
    if x.ndim == 1:
        x = x[:, None]
    xp = jnp.pad(x[:, 0].astype(jnp.float32), (0, NPAD - N))
    ei = edge_index.reshape(-1)
    degp, _ = _deg_kernel(ei, edge_attr)
    deg = degp[:NPAD] + degp[NPAD:]
    dis = jnp.where(deg > 0, deg ** -0.5, 0.0)
    x2 = jnp.concatenate([xp, jnp.zeros((NPAD,), jnp.float32)])
    s1p, _, _, _ = _hop_first(x2, dis, ei, edge_attr)
    s2p, h1, _, _ = _hop_next(s1p, dis, ei, edge_attr)
    s3p, h2, _, _ = _hop_next(s2p, dis, ei, edge_attr)
    out = _head(xp[:, None], h1[:, None], h2[:, None], dis[:, None],
                s3p[:NPAD, None], s3p[NPAD:, None], Wt0.T, Wt1.T, Wt2.T,
                Wt3.T, bt[None, :], W1, b1[None, :], W2.T, b2[None, :])
    return out[:N]


# speedup vs baseline: 287.0223x; 1.1097x over previous
import functools

import jax
import jax.numpy as jnp
from jax import lax
from jax.experimental import pallas as pl
from jax.experimental.pallas import tpu as pltpu
from jax.experimental.pallas import tpu_sc as plsc

N = 100000
E = 6400000
DIM = 128
NPAD = 102400
NSUB = 16
NCORE = 2
NW = NSUB * NCORE
EPW = E // NW
CHUNK = 2000
NCH = EPW // CHUNK
SLC = NPAD // NSUB

_sc_mesh = None


def _mesh():
    global _sc_mesh
    if _sc_mesh is None:
        _sc_mesh = plsc.VectorSubcoreMesh(core_axis_name="c", subcore_axis_name="s")
    return _sc_mesh


_SC_PARAMS = pltpu.CompilerParams(needs_layout_passes=False)


def _zero_vmem(buf, nwords):
    z = jnp.zeros((16,), jnp.float32)

    def zz(i, _):
        o = i * 128
        for k in range(8):
            buf[pl.ds(o + k * 16, 16)] = z
        return 0

    lax.fori_loop(0, nwords // 128, zz, 0)


def _in_copies(ei, ei_off, src_hbm, idxbuf, valbuf, sem, wid, c, b):
    base = pl.multiple_of(wid * EPW + c * CHUNK, 8)
    sl = pl.ds(b * CHUNK, CHUNK)
    return (
        pltpu.make_async_copy(ei.at[pl.ds(ei_off + base, CHUNK)], idxbuf.at[sl],
                              sem.at[b]),
        pltpu.make_async_copy(src_hbm.at[pl.ds(base, CHUNK)], valbuf.at[sl],
                              sem.at[b]),
    )


def _pipe_scatter(acc, ei, src_hbm, idxbuf, valbuf, sem, wid):
    for d in _in_copies(ei, E, src_hbm, idxbuf, valbuf, sem, wid, 0, 0):
        d.start()

    def body(c, _):
        b = lax.rem(c, 2)

        @pl.when(c + 1 < NCH)
        def _():
            for d in _in_copies(ei, E, src_hbm, idxbuf, valbuf, sem, wid,
                                c + 1, 1 - b):
                d.start()

        for d in _in_copies(ei, E, src_hbm, idxbuf, valbuf, sem, wid, c, b):
            d.wait()

        def inner(i, _):
            o = b * CHUNK + i * 80
            for k in range(5):
                ok = o + k * 16
                plsc.addupdate_scatter(acc, [idxbuf[pl.ds(ok, 16)]],
                                       valbuf[pl.ds(ok, 16)])
            return 0

        lax.fori_loop(0, CHUNK // 80, inner, 0)
        return 0

    lax.fori_loop(0, NCH, body, 0)


def _pipe_gather(ubuf, ei, w, msg, idxbuf, valbuf, msgbuf, semi, semo, wid):
    def out_copy(c, b):
        base = pl.multiple_of(wid * EPW + c * CHUNK, 8)
        return pltpu.make_async_copy(msgbuf.at[pl.ds(b * CHUNK, CHUNK)],
                                     msg.at[pl.ds(base, CHUNK)], semo.at[b])

    for d in _in_copies(ei, 0, w, idxbuf, valbuf, semi, wid, 0, 0):
        d.start()

    def body(c, _):
        b = lax.rem(c, 2)

        @pl.when(c + 1 < NCH)
        def _():
            for d in _in_copies(ei, 0, w, idxbuf, valbuf, semi, wid, c + 1, 1 - b):
                d.start()

        for d in _in_copies(ei, 0, w, idxbuf, valbuf, semi, wid, c, b):
            d.wait()

        @pl.when(c >= 2)
        def _():
            out_copy(c - 2, b).wait()

        def inner(i, _):
            o = b * CHUNK + i * 80
            for k in range(5):
                ok = o + k * 16
                g = plsc.load_gather(ubuf, [idxbuf[pl.ds(ok, 16)]])
                msgbuf[pl.ds(ok, 16)] = g * valbuf[pl.ds(ok, 16)]
            return 0

        lax.fori_loop(0, CHUNK // 80, inner, 0)
        out_copy(c, b).start()
        return 0

    lax.fori_loop(0, NCH, body, 0)
    out_copy(NCH - 2, 0).wait()
    out_copy(NCH - 1, 1).wait()


def _reduce_to_partial(acc, accs_hbm, out_hbm, cid, sid, semr):
    wid = cid * NSUB + sid
    pltpu.sync_copy(acc, accs_hbm.at[pl.ds(wid * NPAD, NPAD)])
    plsc.subcore_barrier()

    def rd_copy(t):
        src = pl.multiple_of((cid * NSUB + t) * NPAD + sid * SLC, 8)
        return pltpu.make_async_copy(accs_hbm.at[pl.ds(src, SLC)],
                                     acc.at[pl.ds(t * SLC, SLC)], semr)

    def fire(t, _):
        rd_copy(t).start()
        return 0

    def drain(t, _):
        rd_copy(t).wait()
        return 0

    lax.fori_loop(0, NSUB, fire, 0)
    lax.fori_loop(0, NSUB, drain, 0)

    def red(j, _):
        o = j * 16
        v = [acc[pl.ds(t * SLC + o, 16)] for t in range(NSUB)]
        while len(v) > 1:
            v = [v[i] + v[i + 1] for i in range(0, len(v), 2)]
        acc[pl.ds(o, 16)] = v[0]
        return 0

    lax.fori_loop(0, SLC // 16, red, 0)
    pltpu.sync_copy(acc.at[pl.ds(0, SLC)],
                    out_hbm.at[pl.ds(cid * NPAD + sid * SLC, SLC)])


@functools.partial(
    pl.kernel,
    out_type=(
        jax.ShapeDtypeStruct((NCORE * NPAD,), jnp.float32),
        jax.ShapeDtypeStruct((NW * NPAD,), jnp.float32),
    ),
    mesh=_mesh(),
    compiler_params=_SC_PARAMS,
    scratch_types=[
        pltpu.VMEM((NPAD,), jnp.float32),
        pltpu.VMEM((2 * CHUNK,), jnp.int32),
        pltpu.VMEM((2 * CHUNK,), jnp.float32),
        pltpu.SemaphoreType.DMA((2,)),
        pltpu.SemaphoreType.DMA,
    ],
)
def _deg_kernel(ei, w, degp, accs, acc, idxbuf, valbuf, semi, semr):
    cid = lax.axis_index("c")
    sid = lax.axis_index("s")
    wid = cid * NSUB + sid
    _zero_vmem(acc, NPAD)
    _pipe_scatter(acc, ei, w, idxbuf, valbuf, semi, wid)
    _reduce_to_partial(acc, accs, degp, cid, sid, semr)


def _make_hop_kernel(first):
    @functools.partial(
        pl.kernel,
        out_type=(
            jax.ShapeDtypeStruct((NCORE * NPAD,), jnp.float32),
            jax.ShapeDtypeStruct((NPAD,), jnp.float32),
            jax.ShapeDtypeStruct((E,), jnp.float32),
            jax.ShapeDtypeStruct((NW * NPAD,), jnp.float32),
        ),
        mesh=_mesh(),
        compiler_params=_SC_PARAMS,
        scratch_types=[
            pltpu.VMEM((NPAD,), jnp.float32),
            pltpu.VMEM((2 * CHUNK,), jnp.int32),
            pltpu.VMEM((2 * CHUNK,), jnp.float32),
            pltpu.VMEM((2 * CHUNK,), jnp.float32),
            pltpu.VMEM_SHARED((NPAD,), jnp.float32),
            pltpu.SemaphoreType.DMA((2,)),
            pltpu.SemaphoreType.DMA((2,)),
            pltpu.SemaphoreType.DMA,
        ],
    )
    def _hop(pa, nodevec, ei, w, partials, node_out, msg, accs, ubuf, idxbuf,
             valbuf, msgbuf, ubc, semi, semo, semr):
        cid = lax.axis_index("c")
        sid = lax.axis_index("s")
        wid = cid * NSUB + sid
        slc = pl.ds(sid * SLC, SLC)

        pltpu.sync_copy(pa.at[pl.ds(sid * SLC, SLC)], ubuf.at[pl.ds(0, SLC)])
        pltpu.sync_copy(pa.at[pl.ds(NPAD + sid * SLC, SLC)],
                        ubuf.at[pl.ds(SLC, SLC)])
        pltpu.sync_copy(nodevec.at[slc], ubuf.at[pl.ds(2 * SLC, SLC)])

        def pro(j, _):
            o = j * 16
            v = ubuf[pl.ds(2 * SLC + o, 16)]
            if first:
                h = ubuf[pl.ds(o, 16)]
            else:
                s = ubuf[pl.ds(o, 16)] + ubuf[pl.ds(SLC + o, 16)]
                h = v * s
            ubuf[pl.ds(o, 16)] = h
            ubuf[pl.ds(SLC + o, 16)] = v * h
            return 0

        lax.fori_loop(0, SLC // 16, pro, 0)

        @pl.when(cid == 0)
        def _():
            pltpu.sync_copy(ubuf.at[pl.ds(0, SLC)], node_out.at[slc])

        pltpu.sync_copy(ubuf.at[pl.ds(SLC, SLC)], ubc.at[slc])
        plsc.subcore_barrier()
        pltpu.sync_copy(ubc, ubuf)
        plsc.subcore_barrier()

        _pipe_gather(ubuf, ei, w, msg, idxbuf, valbuf, msgbuf, semi, semo, wid)

        _zero_vmem(ubuf, NPAD)
        _pipe_scatter(ubuf, ei, msg, idxbuf, valbuf, semi, wid)
        _reduce_to_partial(ubuf, accs, partials, cid, sid, semr)

    return _hop


_hop_first = _make_hop_kernel(True)
_hop_next = _make_hop_kernel(False)

BLKR = 2048


def _head_body(x0, h1, h2, dis, s30, s31, wt0, wt1, wt2, wt3, bt, w1, b1,
               w2, b2, out):
    h3 = dis[...] * (s30[...] + s31[...])
    z = x0[...] * wt0[...]
    z = z + h1[...] * wt1[...]
    z = z + h2[...] * wt2[...]
    z = z + h3 * wt3[...]
    z = jnp.maximum(z + bt[...], 0.0)
    z = lax.dot_general(z, w1[...], (((1,), (1,)), ((), ())),
                        precision=lax.Precision.DEFAULT,
                        preferred_element_type=jnp.float32)
    z = jnp.maximum(z + b1[...], 0.0)
    y = lax.dot_general(z, w2[...], (((1,), (0,)), ((), ())),
                        precision=lax.Precision.DEFAULT,
                        preferred_element_type=jnp.float32) + b2[...]
    out[...] = jnp.maximum(y, 0.0)


def _head(x0, h1, h2, dis, s30, s31, wt0, wt1, wt2, wt3, bt, w1, b1, w2, b2):
    full = lambda r, c: pl.BlockSpec((r, c), lambda i: (0, 0))
    blk = lambda c: pl.BlockSpec((BLKR, c), lambda i: (i, 0))
    return pl.pallas_call(
        _head_body,
        grid=(NPAD // BLKR,),
        in_specs=[blk(1), blk(1), blk(1), blk(1), blk(1), blk(1),
                  full(1, DIM), full(1, DIM), full(1, DIM), full(1, DIM),
                  full(1, DIM), full(DIM, DIM), full(1, DIM), full(DIM, 1),
                  full(1, 1)],
        out_specs=blk(1),
        out_shape=jax.ShapeDtypeStruct((NPAD, 1), jnp.float32),
    )(x0, h1, h2, dis, s30, s31, wt0, wt1, wt2, wt3, bt, w1, b1, w2, b2)


def kernel(x, edge_index, edge_attr, Wt0, Wt1, Wt2, Wt3, bt, W1, b1, W2, b2):
    if x.ndim == 1:
        x = x[:, None]
    xp = jnp.pad(x[:, 0].astype(jnp.float32), (0, NPAD - N))
    ei = edge_index.reshape(-1)
    degp, _ = _deg_kernel(ei, edge_attr)
    deg = degp[:NPAD] + degp[NPAD:]
    dis = jnp.where(deg > 0, deg ** -0.5, 0.0)
    x2 = jnp.concatenate([xp, jnp.zeros((NPAD,), jnp.float32)])
    s1p, _, _, _ = _hop_first(x2, dis, ei, edge_attr)
    s2p, h1, _, _ = _hop_next(s1p, dis, ei, edge_attr)
    s3p, h2, _, _ = _hop_next(s2p, dis, ei, edge_attr)
    out = _head(xp[:, None], h1[:, None], h2[:, None], dis[:, None],
                s3p[:NPAD, None], s3p[NPAD:, None], Wt0.T, Wt1.T, Wt2.T,
                Wt3.T, bt[None, :], W1, b1[None, :], W2.T, b2[None, :])
    return out[:N]


# speedup vs baseline: 288.6760x; 1.0058x over previous
import functools

import jax
import jax.numpy as jnp
from jax import lax
from jax.experimental import pallas as pl
from jax.experimental.pallas import tpu as pltpu
from jax.experimental.pallas import tpu_sc as plsc

N = 100000
E = 6400000
DIM = 128
NPAD = 102400
NSUB = 16
NCORE = 2
NW = NSUB * NCORE
EPW = E // NW
CHUNK = 2000
NCH = EPW // CHUNK
SLC = NPAD // NSUB

_sc_mesh = None


def _mesh():
    global _sc_mesh
    if _sc_mesh is None:
        _sc_mesh = plsc.VectorSubcoreMesh(core_axis_name="c", subcore_axis_name="s")
    return _sc_mesh


_SC_PARAMS = pltpu.CompilerParams(needs_layout_passes=False)


def _zero_vmem(buf, nwords):
    z = jnp.zeros((16,), jnp.float32)

    def zz(i, _):
        o = i * 128
        for k in range(8):
            buf[pl.ds(o + k * 16, 16)] = z
        return 0

    lax.fori_loop(0, nwords // 128, zz, 0)


def _in_copies(ei, ei_off, src_hbm, idxbuf, valbuf, sem, wid, c, b):
    base = pl.multiple_of(wid * EPW + c * CHUNK, 8)
    sl = pl.ds(b * CHUNK, CHUNK)
    return (
        pltpu.make_async_copy(ei.at[pl.ds(ei_off + base, CHUNK)], idxbuf.at[sl],
                              sem.at[b]),
        pltpu.make_async_copy(src_hbm.at[pl.ds(base, CHUNK)], valbuf.at[sl],
                              sem.at[b]),
    )


def _pipe_scatter(acc, ei, src_hbm, idxbuf, valbuf, sem, wid):
    for p in range(2):
        for d in _in_copies(ei, E, src_hbm, idxbuf, valbuf, sem, wid, p, p):
            d.start()

    def body(c, _):
        b = lax.rem(c, 3)

        @pl.when(c + 2 < NCH)
        def _():
            for d in _in_copies(ei, E, src_hbm, idxbuf, valbuf, sem, wid,
                                c + 2, lax.rem(c + 2, 3)):
                d.start()

        for d in _in_copies(ei, E, src_hbm, idxbuf, valbuf, sem, wid, c, b):
            d.wait()

        def inner(i, _):
            o = b * CHUNK + i * 80
            for k in range(5):
                ok = o + k * 16
                plsc.addupdate_scatter(acc, [idxbuf[pl.ds(ok, 16)]],
                                       valbuf[pl.ds(ok, 16)])
            return 0

        lax.fori_loop(0, CHUNK // 80, inner, 0)
        return 0

    lax.fori_loop(0, NCH, body, 0)


def _pipe_gather(ubuf, ei, w, msg, idxbuf, valbuf, msgbuf, semi, semo, wid):
    def out_copy(c, b):
        base = pl.multiple_of(wid * EPW + c * CHUNK, 8)
        return pltpu.make_async_copy(msgbuf.at[pl.ds(b * CHUNK, CHUNK)],
                                     msg.at[pl.ds(base, CHUNK)], semo.at[b])

    for p in range(2):
        for d in _in_copies(ei, 0, w, idxbuf, valbuf, semi, wid, p, p):
            d.start()

    def body(c, _):
        b = lax.rem(c, 3)
        m = lax.rem(c, 2)

        @pl.when(c + 2 < NCH)
        def _():
            for d in _in_copies(ei, 0, w, idxbuf, valbuf, semi, wid, c + 2,
                                lax.rem(c + 2, 3)):
                d.start()

        for d in _in_copies(ei, 0, w, idxbuf, valbuf, semi, wid, c, b):
            d.wait()

        @pl.when(c >= 2)
        def _():
            out_copy(c - 2, m).wait()

        def inner(i, _):
            o = b * CHUNK + i * 80
            om = m * CHUNK + i * 80
            for k in range(5):
                ok = o + k * 16
                g = plsc.load_gather(ubuf, [idxbuf[pl.ds(ok, 16)]])
                msgbuf[pl.ds(om + k * 16, 16)] = g * valbuf[pl.ds(ok, 16)]
            return 0

        lax.fori_loop(0, CHUNK // 80, inner, 0)
        out_copy(c, m).start()
        return 0

    lax.fori_loop(0, NCH, body, 0)
    out_copy(NCH - 2, 0).wait()
    out_copy(NCH - 1, 1).wait()


def _reduce_to_partial(acc, accs_hbm, out_hbm, cid, sid, semr):
    wid = cid * NSUB + sid
    pltpu.sync_copy(acc, accs_hbm.at[pl.ds(wid * NPAD, NPAD)])
    plsc.subcore_barrier()

    def rd_copy(t):
        src = pl.multiple_of((cid * NSUB + t) * NPAD + sid * SLC, 8)
        return pltpu.make_async_copy(accs_hbm.at[pl.ds(src, SLC)],
                                     acc.at[pl.ds(t * SLC, SLC)], semr)

    def fire(t, _):
        rd_copy(t).start()
        return 0

    def drain(t, _):
        rd_copy(t).wait()
        return 0

    lax.fori_loop(0, NSUB, fire, 0)
    lax.fori_loop(0, NSUB, drain, 0)

    def red(j, _):
        o = j * 16
        v = [acc[pl.ds(t * SLC + o, 16)] for t in range(NSUB)]
        while len(v) > 1:
            v = [v[i] + v[i + 1] for i in range(0, len(v), 2)]
        acc[pl.ds(o, 16)] = v[0]
        return 0

    lax.fori_loop(0, SLC // 16, red, 0)
    pltpu.sync_copy(acc.at[pl.ds(0, SLC)],
                    out_hbm.at[pl.ds(cid * NPAD + sid * SLC, SLC)])


@functools.partial(
    pl.kernel,
    out_type=(
        jax.ShapeDtypeStruct((NCORE * NPAD,), jnp.float32),
        jax.ShapeDtypeStruct((NW * NPAD,), jnp.float32),
    ),
    mesh=_mesh(),
    compiler_params=_SC_PARAMS,
    scratch_types=[
        pltpu.VMEM((NPAD,), jnp.float32),
        pltpu.VMEM((3 * CHUNK,), jnp.int32),
        pltpu.VMEM((3 * CHUNK,), jnp.float32),
        pltpu.SemaphoreType.DMA((3,)),
        pltpu.SemaphoreType.DMA,
    ],
)
def _deg_kernel(ei, w, degp, accs, acc, idxbuf, valbuf, semi, semr):
    cid = lax.axis_index("c")
    sid = lax.axis_index("s")
    wid = cid * NSUB + sid
    _zero_vmem(acc, NPAD)
    _pipe_scatter(acc, ei, w, idxbuf, valbuf, semi, wid)
    _reduce_to_partial(acc, accs, degp, cid, sid, semr)


def _make_hop_kernel(first):
    @functools.partial(
        pl.kernel,
        out_type=(
            jax.ShapeDtypeStruct((NCORE * NPAD,), jnp.float32),
            jax.ShapeDtypeStruct((NPAD,), jnp.float32),
            jax.ShapeDtypeStruct((E,), jnp.float32),
            jax.ShapeDtypeStruct((NW * NPAD,), jnp.float32),
        ),
        mesh=_mesh(),
        compiler_params=_SC_PARAMS,
        scratch_types=[
            pltpu.VMEM((NPAD,), jnp.float32),
            pltpu.VMEM((3 * CHUNK,), jnp.int32),
            pltpu.VMEM((3 * CHUNK,), jnp.float32),
            pltpu.VMEM((2 * CHUNK,), jnp.float32),
            pltpu.VMEM_SHARED((NPAD,), jnp.float32),
            pltpu.SemaphoreType.DMA((3,)),
            pltpu.SemaphoreType.DMA((2,)),
            pltpu.SemaphoreType.DMA,
        ],
    )
    def _hop(pa, nodevec, ei, w, partials, node_out, msg, accs, ubuf, idxbuf,
             valbuf, msgbuf, ubc, semi, semo, semr):
        cid = lax.axis_index("c")
        sid = lax.axis_index("s")
        wid = cid * NSUB + sid
        slc = pl.ds(sid * SLC, SLC)

        pltpu.sync_copy(pa.at[pl.ds(sid * SLC, SLC)], ubuf.at[pl.ds(0, SLC)])
        pltpu.sync_copy(pa.at[pl.ds(NPAD + sid * SLC, SLC)],
                        ubuf.at[pl.ds(SLC, SLC)])
        pltpu.sync_copy(nodevec.at[slc], ubuf.at[pl.ds(2 * SLC, SLC)])

        def pro(j, _):
            o = j * 16
            v = ubuf[pl.ds(2 * SLC + o, 16)]
            if first:
                h = ubuf[pl.ds(o, 16)]
            else:
                s = ubuf[pl.ds(o, 16)] + ubuf[pl.ds(SLC + o, 16)]
                h = v * s
            ubuf[pl.ds(o, 16)] = h
            ubuf[pl.ds(SLC + o, 16)] = v * h
            return 0

        lax.fori_loop(0, SLC // 16, pro, 0)

        @pl.when(cid == 0)
        def _():
            pltpu.sync_copy(ubuf.at[pl.ds(0, SLC)], node_out.at[slc])

        pltpu.sync_copy(ubuf.at[pl.ds(SLC, SLC)], ubc.at[slc])
        plsc.subcore_barrier()
        pltpu.sync_copy(ubc, ubuf)
        plsc.subcore_barrier()

        _pipe_gather(ubuf, ei, w, msg, idxbuf, valbuf, msgbuf, semi, semo, wid)

        _zero_vmem(ubuf, NPAD)
        _pipe_scatter(ubuf, ei, msg, idxbuf, valbuf, semi, wid)
        _reduce_to_partial(ubuf, accs, partials, cid, sid, semr)

    return _hop


_hop_first = _make_hop_kernel(True)
_hop_next = _make_hop_kernel(False)

BLKR = 2048


def _head_body(x0, h1, h2, dis, s30, s31, wt0, wt1, wt2, wt3, bt, w1, b1,
               w2, b2, out):
    h3 = dis[...] * (s30[...] + s31[...])
    z = x0[...] * wt0[...]
    z = z + h1[...] * wt1[...]
    z = z + h2[...] * wt2[...]
    z = z + h3 * wt3[...]
    z = jnp.maximum(z + bt[...], 0.0)
    z = lax.dot_general(z, w1[...], (((1,), (1,)), ((), ())),
                        precision=lax.Precision.DEFAULT,
                        preferred_element_type=jnp.float32)
    z = jnp.maximum(z + b1[...], 0.0)
    y = lax.dot_general(z, w2[...], (((1,), (0,)), ((), ())),
                        precision=lax.Precision.DEFAULT,
                        preferred_element_type=jnp.float32) + b2[...]
    out[...] = jnp.maximum(y, 0.0)


def _head(x0, h1, h2, dis, s30, s31, wt0, wt1, wt2, wt3, bt, w1, b1, w2, b2):
    full = lambda r, c: pl.BlockSpec((r, c), lambda i: (0, 0))
    blk = lambda c: pl.BlockSpec((BLKR, c), lambda i: (i, 0))
    return pl.pallas_call(
        _head_body,
        grid=(NPAD // BLKR,),
        in_specs=[blk(1), blk(1), blk(1), blk(1), blk(1), blk(1),
                  full(1, DIM), full(1, DIM), full(1, DIM), full(1, DIM),
                  full(1, DIM), full(DIM, DIM), full(1, DIM), full(DIM, 1),
                  full(1, 1)],
        out_specs=blk(1),
        out_shape=jax.ShapeDtypeStruct((NPAD, 1), jnp.float32),
    )(x0, h1, h2, dis, s30, s31, wt0, wt1, wt2, wt3, bt, w1, b1, w2, b2)


def kernel(x, edge_index, edge_attr, Wt0, Wt1, Wt2, Wt3, bt, W1, b1, W2, b2):
    if x.ndim == 1:
        x = x[:, None]
    xp = jnp.pad(x[:, 0].astype(jnp.float32), (0, NPAD - N))
    ei = edge_index.reshape(-1)
    degp, _ = _deg_kernel(ei, edge_attr)
    deg = degp[:NPAD] + degp[NPAD:]
    dis = jnp.where(deg > 0, deg ** -0.5, 0.0)
    x2 = jnp.concatenate([xp, jnp.zeros((NPAD,), jnp.float32)])
    s1p, _, _, _ = _hop_first(x2, dis, ei, edge_attr)
    s2p, h1, _, _ = _hop_next(s1p, dis, ei, edge_attr)
    s3p, h2, _, _ = _hop_next(s2p, dis, ei, edge_attr)
    out = _head(xp[:, None], h1[:, None], h2[:, None], dis[:, None],
                s3p[:NPAD, None], s3p[NPAD:, None], Wt0.T, Wt1.T, Wt2.T,
                Wt3.T, bt[None, :], W1, b1[None, :], W2.T, b2[None, :])
    return out[:N]
